# Initial kernel scaffold; baseline (speedup 1.0000x reference)
import jax
import jax.numpy as jnp
from jax.experimental import pallas as pl


def kernel(x1, x4, edge_index, W_ih, W_hh, b_ih, b_hh, W1, b1, W2, b2, W3, b3, W4, b4, W7, b7):
    raise NotImplementedError("write your pallas kernel here")


# speedup vs baseline: 2.4444x; 2.4444x over previous
import jax
import jax.numpy as jnp
from jax.experimental import pallas as pl
from jax.experimental.pallas import tpu as pltpu

H3 = 384
FEAT = 256
T = 8
BLOCK_N = 400


def _gru_body(x1_ref, x4_ref, wih_ref, whh_ref, brz_ref, bin_ref, bhn_ref,
              d1_ref, d4_ref, dc_ref, out_ref):
    wih = wih_ref[...]
    whh = whh_ref[...]
    brz = brz_ref[...]
    b_in = bin_ref[...]
    b_hn = bhn_ref[...]

    h = None
    for t in range(T):
        xt = x1_ref[:, t, :]
        gi = jnp.dot(xt, wih, preferred_element_type=jnp.float32)
        if h is None:
            gh = jnp.zeros_like(gi)
        else:
            gh = jnp.dot(h.astype(jnp.bfloat16), whh,
                         preferred_element_type=jnp.float32)
        rz = jax.nn.sigmoid(gi[:, :2 * H3] + gh[:, :2 * H3] + brz)
        r = rz[:, :H3]
        z = rz[:, H3:]
        n = jnp.tanh(gi[:, 2 * H3:] + b_in + r * (gh[:, 2 * H3:] + b_hn))
        if h is None:
            h = (1.0 - z) * n
        else:
            h = (1.0 - z) * n + z * h

    delta = jnp.sum(h * d1_ref[...], axis=1, keepdims=True)
    delta = delta + jnp.sum(x4_ref[...] * d4_ref[...], axis=1, keepdims=True)
    delta = delta + dc_ref[0, 0]
    p0 = jax.nn.sigmoid(delta)
    out_ref[:, 0:1] = p0
    out_ref[:, 1:2] = 1.0 - p0


def kernel(x1, x4, edge_index, W_ih, W_hh, b_ih, b_hh,
           W1, b1, W2, b2, W3, b3, W4, b4, W7, b7):
    n = x1.shape[0]
    ccl = x4.shape[1]

    w7 = W7[0] - W7[1]
    d1 = W1.T @ w7[:128]
    d4 = W4.T @ w7[384:]
    dc = (b1 @ w7[:128] + b2 @ w7[128:256] + b3 @ w7[256:384]
          + b4 @ w7[384:] + (b7[0] - b7[1]))

    grid = (n // BLOCK_N,)
    out = pl.pallas_call(
        _gru_body,
        grid=grid,
        in_specs=[
            pl.BlockSpec((BLOCK_N, T, FEAT), lambda i: (i, 0, 0)),
            pl.BlockSpec((BLOCK_N, ccl), lambda i: (i, 0)),
            pl.BlockSpec((FEAT, 3 * H3), lambda i: (0, 0)),
            pl.BlockSpec((H3, 3 * H3), lambda i: (0, 0)),
            pl.BlockSpec((1, 2 * H3), lambda i: (0, 0)),
            pl.BlockSpec((1, H3), lambda i: (0, 0)),
            pl.BlockSpec((1, H3), lambda i: (0, 0)),
            pl.BlockSpec((1, H3), lambda i: (0, 0)),
            pl.BlockSpec((1, ccl), lambda i: (0, 0)),
            pl.BlockSpec((1, 1), lambda i: (0, 0)),
        ],
        out_specs=pl.BlockSpec((BLOCK_N, 2), lambda i: (i, 0)),
        out_shape=jax.ShapeDtypeStruct((n, 2), jnp.float32),
        compiler_params=pltpu.CompilerParams(
            dimension_semantics=("arbitrary",)),
    )(
        x1.astype(jnp.bfloat16),
        x4,
        W_ih.T.astype(jnp.bfloat16),
        W_hh.T.astype(jnp.bfloat16),
        (b_ih[:2 * H3] + b_hh[:2 * H3])[None, :],
        b_ih[None, 2 * H3:],
        b_hh[None, 2 * H3:],
        d1[None, :],
        d4[None, :],
        dc.reshape(1, 1),
    )
    return out.T


# speedup vs baseline: 2.9163x; 1.1931x over previous
import jax
import jax.numpy as jnp
from jax.experimental import pallas as pl
from jax.experimental.pallas import tpu as pltpu

H3 = 384
FEAT = 256
T = 8
BLOCK_N = 400


def _gru_body(x1_ref, x4_ref, wih_ref, whh_ref, brz_ref, bin_ref, bhn_ref,
              d1_ref, d4_ref, dc_ref, out_ref):
    whh = whh_ref[...]
    brz = brz_ref[...]
    b_in = bin_ref[...]
    b_hn = bhn_ref[...]

    xall = x1_ref[...].reshape(T * BLOCK_N, FEAT)
    gi_all = jnp.dot(xall, wih_ref[...],
                     preferred_element_type=jnp.float32
                     ).reshape(T, BLOCK_N, 3 * H3)

    h = None
    for t in range(T):
        gi = gi_all[t]
        if h is None:
            gh_rz = brz
            gh_n = b_hn
        else:
            gh = jnp.dot(h.astype(jnp.bfloat16), whh,
                         preferred_element_type=jnp.float32)
            gh_rz = gh[:, :2 * H3] + brz
            gh_n = gh[:, 2 * H3:] + b_hn
        rz = jax.nn.sigmoid(gi[:, :2 * H3] + gh_rz)
        r = rz[:, :H3]
        z = rz[:, H3:]
        n = jnp.tanh(gi[:, 2 * H3:] + b_in + r * gh_n)
        if h is None:
            h = n - z * n
        else:
            h = n + z * (h - n)

    delta = jnp.sum(h * d1_ref[...], axis=1, keepdims=True)
    delta = delta + jnp.sum(x4_ref[...] * d4_ref[...], axis=1, keepdims=True)
    delta = delta + dc_ref[0, 0]
    p0 = jax.nn.sigmoid(delta)
    out_ref[:, 0:1] = p0
    out_ref[:, 1:2] = 1.0 - p0


def kernel(x1, x4, edge_index, W_ih, W_hh, b_ih, b_hh,
           W1, b1, W2, b2, W3, b3, W4, b4, W7, b7):
    n = x1.shape[0]
    ccl = x4.shape[1]

    w7 = W7[0] - W7[1]
    d1 = W1.T @ w7[:128]
    d4 = W4.T @ w7[384:]
    dc = (b1 @ w7[:128] + b2 @ w7[128:256] + b3 @ w7[256:384]
          + b4 @ w7[384:] + (b7[0] - b7[1]))

    grid = (n // BLOCK_N,)
    out = pl.pallas_call(
        _gru_body,
        grid=grid,
        in_specs=[
            pl.BlockSpec((T, BLOCK_N, FEAT), lambda i: (0, i, 0)),
            pl.BlockSpec((BLOCK_N, ccl), lambda i: (i, 0)),
            pl.BlockSpec((FEAT, 3 * H3), lambda i: (0, 0)),
            pl.BlockSpec((H3, 3 * H3), lambda i: (0, 0)),
            pl.BlockSpec((1, 2 * H3), lambda i: (0, 0)),
            pl.BlockSpec((1, H3), lambda i: (0, 0)),
            pl.BlockSpec((1, H3), lambda i: (0, 0)),
            pl.BlockSpec((1, H3), lambda i: (0, 0)),
            pl.BlockSpec((1, ccl), lambda i: (0, 0)),
            pl.BlockSpec((1, 1), lambda i: (0, 0)),
        ],
        out_specs=pl.BlockSpec((BLOCK_N, 2), lambda i: (i, 0)),
        out_shape=jax.ShapeDtypeStruct((n, 2), jnp.float32),
        compiler_params=pltpu.CompilerParams(
            dimension_semantics=("arbitrary",)),
    )(
        jnp.swapaxes(x1, 0, 1).astype(jnp.bfloat16),
        x4,
        W_ih.T.astype(jnp.bfloat16),
        W_hh.T.astype(jnp.bfloat16),
        (b_ih[:2 * H3] + b_hh[:2 * H3])[None, :],
        b_ih[None, 2 * H3:],
        b_hh[None, 2 * H3:],
        d1[None, :],
        d4[None, :],
        dc.reshape(1, 1),
    )
    return out.T


# speedup vs baseline: 2.9225x; 1.0021x over previous
import jax
import jax.numpy as jnp
from jax.experimental import pallas as pl
from jax.experimental.pallas import tpu as pltpu

H3 = 384
FEAT = 256
T = 8
BLOCK_N = 400


def _sigmoid(x):
    return 0.5 * jnp.tanh(0.5 * x) + 0.5


def _gru_step(gi, h, whh, brz, b_in, b_hn):
    if h is None:
        gh_rz = brz
        gh_n = b_hn
    else:
        gh = jnp.dot(h.astype(jnp.bfloat16), whh,
                     preferred_element_type=jnp.float32)
        gh_rz = gh[:, :2 * H3] + brz
        gh_n = gh[:, 2 * H3:] + b_hn
    rz = _sigmoid(gi[:, :2 * H3] + gh_rz)
    r = rz[:, :H3]
    z = rz[:, H3:]
    n = jnp.tanh(gi[:, 2 * H3:] + b_in + r * gh_n)
    if h is None:
        return n - z * n
    return n + z * (h - n)


def _gru_body(x1_ref, x4_ref, wih_ref, whh_ref, brz_ref, bin_ref, bhn_ref,
              d1_ref, d4_ref, dc_ref, out_ref):
    whh = whh_ref[...]
    brz = brz_ref[...]
    b_in = bin_ref[...]
    b_hn = bhn_ref[...]

    xall = x1_ref[...].reshape(T * BLOCK_N, FEAT)
    gi_all = jnp.dot(xall, wih_ref[...],
                     preferred_element_type=jnp.float32
                     ).reshape(T, BLOCK_N, 3 * H3)

    hb = BLOCK_N // 2
    hs = [None, None]
    for t in range(T):
        gi = gi_all[t]
        for k in range(2):
            hs[k] = _gru_step(gi[k * hb:(k + 1) * hb], hs[k],
                              whh, brz, b_in, b_hn)

    h = jnp.concatenate(hs, axis=0)
    delta = jnp.sum(h * d1_ref[...], axis=1, keepdims=True)
    delta = delta + jnp.sum(x4_ref[...] * d4_ref[...], axis=1, keepdims=True)
    delta = delta + dc_ref[0, 0]
    p0 = _sigmoid(delta)
    out_ref[:, 0:1] = p0
    out_ref[:, 1:2] = 1.0 - p0


def kernel(x1, x4, edge_index, W_ih, W_hh, b_ih, b_hh,
           W1, b1, W2, b2, W3, b3, W4, b4, W7, b7):
    n = x1.shape[0]
    ccl = x4.shape[1]

    w7 = W7[0] - W7[1]
    d1 = W1.T @ w7[:128]
    d4 = W4.T @ w7[384:]
    dc = (b1 @ w7[:128] + b2 @ w7[128:256] + b3 @ w7[256:384]
          + b4 @ w7[384:] + (b7[0] - b7[1]))

    grid = (n // BLOCK_N,)
    out = pl.pallas_call(
        _gru_body,
        grid=grid,
        in_specs=[
            pl.BlockSpec((T, BLOCK_N, FEAT), lambda i: (0, i, 0)),
            pl.BlockSpec((BLOCK_N, ccl), lambda i: (i, 0)),
            pl.BlockSpec((FEAT, 3 * H3), lambda i: (0, 0)),
            pl.BlockSpec((H3, 3 * H3), lambda i: (0, 0)),
            pl.BlockSpec((1, 2 * H3), lambda i: (0, 0)),
            pl.BlockSpec((1, H3), lambda i: (0, 0)),
            pl.BlockSpec((1, H3), lambda i: (0, 0)),
            pl.BlockSpec((1, H3), lambda i: (0, 0)),
            pl.BlockSpec((1, ccl), lambda i: (0, 0)),
            pl.BlockSpec((1, 1), lambda i: (0, 0)),
        ],
        out_specs=pl.BlockSpec((BLOCK_N, 2), lambda i: (i, 0)),
        out_shape=jax.ShapeDtypeStruct((n, 2), jnp.float32),
        compiler_params=pltpu.CompilerParams(
            dimension_semantics=("arbitrary",)),
    )(
        jnp.swapaxes(x1, 0, 1).astype(jnp.bfloat16),
        x4,
        W_ih.T.astype(jnp.bfloat16),
        W_hh.T.astype(jnp.bfloat16),
        (b_ih[:2 * H3] + b_hh[:2 * H3])[None, :],
        b_ih[None, 2 * H3:],
        b_hh[None, 2 * H3:],
        d1[None, :],
        d4[None, :],
        dc.reshape(1, 1),
    )
    return out.T


# speedup vs baseline: 2.9950x; 1.0248x over previous
import jax
import jax.numpy as jnp
from jax.experimental import pallas as pl
from jax.experimental.pallas import tpu as pltpu

H3 = 384
FEAT = 256
T = 8
BLOCK_N = 1000


def _sigmoid(x):
    return 0.5 * jnp.tanh(0.5 * x) + 0.5


def _gru_step(gi, h, whh, brz, b_in, b_hn):
    if h is None:
        gh_rz = brz
        gh_n = b_hn
    else:
        gh = jnp.dot(h.astype(jnp.bfloat16), whh,
                     preferred_element_type=jnp.float32)
        gh_rz = gh[:, :2 * H3] + brz
        gh_n = gh[:, 2 * H3:] + b_hn
    rz = _sigmoid(gi[:, :2 * H3] + gh_rz)
    r = rz[:, :H3]
    z = rz[:, H3:]
    n = jnp.tanh(gi[:, 2 * H3:] + b_in + r * gh_n)
    if h is None:
        return n - z * n
    return n + z * (h - n)


def _gru_body(x1_ref, x4_ref, wih_ref, whh_ref, brz_ref, bin_ref, bhn_ref,
              d1_ref, d4_ref, dc_ref, out_ref):
    whh = whh_ref[...]
    brz = brz_ref[...]
    b_in = bin_ref[...]
    b_hn = bhn_ref[...]

    xall = x1_ref[...].reshape(T * BLOCK_N, FEAT)
    gi_all = jnp.dot(xall, wih_ref[...],
                     preferred_element_type=jnp.float32
                     ).reshape(T, BLOCK_N, 3 * H3)

    nchains = 5
    hb = BLOCK_N // nchains
    hs = [None] * nchains
    for t in range(T):
        gi = gi_all[t]
        for k in range(nchains):
            hs[k] = _gru_step(gi[k * hb:(k + 1) * hb], hs[k],
                              whh, brz, b_in, b_hn)

    h = jnp.concatenate(hs, axis=0)
    delta = jnp.sum(h * d1_ref[...], axis=1, keepdims=True)
    delta = delta + jnp.sum(x4_ref[...] * d4_ref[...], axis=1, keepdims=True)
    delta = delta + dc_ref[0, 0]
    p0 = _sigmoid(delta)
    out_ref[:, 0:1] = p0
    out_ref[:, 1:2] = 1.0 - p0


def kernel(x1, x4, edge_index, W_ih, W_hh, b_ih, b_hh,
           W1, b1, W2, b2, W3, b3, W4, b4, W7, b7):
    n = x1.shape[0]
    ccl = x4.shape[1]

    w7 = W7[0] - W7[1]
    d1 = W1.T @ w7[:128]
    d4 = W4.T @ w7[384:]
    dc = (b1 @ w7[:128] + b2 @ w7[128:256] + b3 @ w7[256:384]
          + b4 @ w7[384:] + (b7[0] - b7[1]))

    grid = (n // BLOCK_N,)
    out = pl.pallas_call(
        _gru_body,
        grid=grid,
        in_specs=[
            pl.BlockSpec((T, BLOCK_N, FEAT), lambda i: (0, i, 0)),
            pl.BlockSpec((BLOCK_N, ccl), lambda i: (i, 0)),
            pl.BlockSpec((FEAT, 3 * H3), lambda i: (0, 0)),
            pl.BlockSpec((H3, 3 * H3), lambda i: (0, 0)),
            pl.BlockSpec((1, 2 * H3), lambda i: (0, 0)),
            pl.BlockSpec((1, H3), lambda i: (0, 0)),
            pl.BlockSpec((1, H3), lambda i: (0, 0)),
            pl.BlockSpec((1, H3), lambda i: (0, 0)),
            pl.BlockSpec((1, ccl), lambda i: (0, 0)),
            pl.BlockSpec((1, 1), lambda i: (0, 0)),
        ],
        out_specs=pl.BlockSpec((BLOCK_N, 2), lambda i: (i, 0)),
        out_shape=jax.ShapeDtypeStruct((n, 2), jnp.float32),
        compiler_params=pltpu.CompilerParams(
            dimension_semantics=("arbitrary",)),
    )(
        jnp.swapaxes(x1, 0, 1).astype(jnp.bfloat16),
        x4,
        W_ih.T.astype(jnp.bfloat16),
        W_hh.T.astype(jnp.bfloat16),
        (b_ih[:2 * H3] + b_hh[:2 * H3])[None, :],
        b_ih[None, 2 * H3:],
        b_hh[None, 2 * H3:],
        d1[None, :],
        d4[None, :],
        dc.reshape(1, 1),
    )
    return out.T
